# Initial kernel scaffold; baseline (speedup 1.0000x reference)
import functools

import jax
import jax.numpy as jnp
from jax import lax
from jax.experimental import pallas as pl
from jax.experimental.pallas import tpu as pltpu
from jax.experimental.pallas import tpu_sc as plsc

N = 10000
DF = 128
R = 2
B = 1024
E = 160000
TOPK = E // 2

NC, NS, L = 2, 16, 16
EPT = E // NS
BPT = B // NS
G = 80
NCHUNK = EPT // G
NPAD = 10240
DUMMY = N

_mesh = plsc.VectorSubcoreMesh(
    core_axis_name="c", subcore_axis_name="s", num_cores=NC, num_subcores=NS)


def _dnet_body(f_ref, W1_ref, b1_ref, W2_ref, b2_ref, d_ref):
    f = f_ref[...]
    for r in range(R):
        h = jnp.dot(f, W1_ref[r], preferred_element_type=jnp.float32)
        h = jnp.maximum(h + b1_ref[r][None, :], 0.0)
        z = jnp.dot(h, W2_ref[r, :, 1], preferred_element_type=jnp.float32)
        d_ref[r] = jax.nn.sigmoid(z + b2_ref[r, 1])


def _dnet(features, W1, b1, W2, b2):
    return pl.pallas_call(
        _dnet_body,
        out_shape=jax.ShapeDtypeStruct((R, N), jnp.float32),
    )(features, W1, b1, W2, b2)


@functools.partial(
    pl.kernel,
    out_type=[jax.ShapeDtypeStruct((R, E), jnp.float32),
              jax.ShapeDtypeStruct((R, B, DF), jnp.float32)],
    mesh=_mesh,
    scratch_types=[pltpu.VMEM((N,), jnp.float32),
                   pltpu.VMEM((EPT,), jnp.int32),
                   pltpu.VMEM((EPT,), jnp.int32),
                   pltpu.VMEM((EPT,), jnp.float32),
                   pltpu.VMEM((BPT,), jnp.int32),
                   pltpu.VMEM((BPT, DF), jnp.float32),
                   pltpu.SemaphoreType.DMA],
)
def _k2(d_hbm, edge_hbm, picked_hbm, feat_hbm, diffs_hbm, ctr_hbm,
        d_v, src_v, dst_v, diff_v, pick_v, rows_v, sem):
    c = lax.axis_index("c")
    s = lax.axis_index("s")
    base = s * EPT
    pltpu.sync_copy(d_hbm.at[c], d_v)
    pltpu.sync_copy(edge_hbm.at[c, 0, pl.ds(base, EPT)], src_v)
    pltpu.sync_copy(edge_hbm.at[c, 1, pl.ds(base, EPT)], dst_v)

    def body(i, carry):
        si = src_v[pl.ds(i * L, L)]
        di = dst_v[pl.ds(i * L, L)]
        a = plsc.load_gather(d_v, [si])
        b = plsc.load_gather(d_v, [di])
        diff_v[pl.ds(i * L, L)] = jnp.abs(b - a)
        return carry

    lax.fori_loop(0, EPT // L, body, 0)
    pltpu.sync_copy(diff_v, diffs_hbm.at[c, pl.ds(base, EPT)])

    pltpu.sync_copy(picked_hbm.at[c, pl.ds(s * BPT, BPT)], pick_v)
    pltpu.async_copy(feat_hbm.at[pick_v], rows_v, sem).wait()
    pltpu.sync_copy(rows_v, ctr_hbm.at[c, pl.ds(s * BPT, BPT)])


def _rho_body(diffs_ref, rho_ref):
    d = diffs_ref[...]
    bits = lax.bitcast_convert_type(d, jnp.int32)

    def step(i, lohi):
        lo, hi = lohi
        mid = (lo + hi + 1) // 2
        cnt = jnp.sum((bits >= mid).astype(jnp.int32), axis=1, keepdims=True)
        take = cnt >= TOPK
        return jnp.where(take, mid, lo), jnp.where(take, hi, mid - 1)

    lo0 = jnp.zeros((R, 1), jnp.int32)
    hi0 = jnp.full((R, 1), 0x3F800000, jnp.int32)
    lo, _ = lax.fori_loop(0, 31, step, (lo0, hi0))
    t = lax.bitcast_convert_type(lo, jnp.float32)
    gt = d > t
    cnt_gt = jnp.sum(gt.astype(jnp.float32), axis=1, keepdims=True)
    sum_gt = jnp.sum(jnp.where(gt, d, 0.0), axis=1, keepdims=True)
    rho = (sum_gt + t * (TOPK - cnt_gt)) * (1.0 / TOPK)
    rho_ref[...] = jnp.broadcast_to(rho, (R, 128))


def _rho(diffs):
    return pl.pallas_call(
        _rho_body,
        out_shape=jax.ShapeDtypeStruct((R, 128), jnp.float32),
    )(diffs)


@functools.partial(
    pl.kernel,
    out_type=[jax.ShapeDtypeStruct((R, B, DF), jnp.float32)],
    mesh=_mesh,
    scratch_types=[pltpu.VMEM((EPT,), jnp.int32),
                   pltpu.VMEM((EPT,), jnp.int32),
                   pltpu.VMEM((EPT,), jnp.float32),
                   pltpu.VMEM((L,), jnp.float32),
                   pltpu.VMEM((BPT,), jnp.int32),
                   pltpu.VMEM((G,), jnp.int32),
                   pltpu.VMEM((G, DF), jnp.float32),
                   pltpu.VMEM((64, DF), jnp.float32),
                   pltpu.VMEM((BPT, DF), jnp.float32),
                   pltpu.VMEM_SHARED((NPAD, DF), jnp.float32),
                   pltpu.SemaphoreType.DMA],
)
def _k4(rho_hbm, diffs_hbm, edge_hbm, picked_hbm, feat_hbm, neigh_hbm,
        src_v, dst_v, diff_v, rho_v, pick_v, idx_v, rows_v, zrows_v,
        out_v, acc, sem):
    c = lax.axis_index("c")
    s = lax.axis_index("s")
    base = s * EPT

    pltpu.sync_copy(edge_hbm.at[c, 0, pl.ds(base, EPT)], src_v)
    pltpu.sync_copy(edge_hbm.at[c, 1, pl.ds(base, EPT)], dst_v)
    pltpu.sync_copy(diffs_hbm.at[c, pl.ds(base, EPT)], diff_v)
    pltpu.sync_copy(rho_hbm.at[c, pl.ds(0, L)], rho_v)
    pltpu.sync_copy(picked_hbm.at[c, pl.ds(s * BPT, BPT)], pick_v)

    def zb(i, carry):
        def zb2(j, carry2):
            zrows_v[i, pl.ds(j * L, L)] = jnp.zeros((L,), jnp.float32)
            return carry2
        return lax.fori_loop(0, DF // L, zb2, carry)
    lax.fori_loop(0, 64, zb, 0)
    for j in range(NPAD // NS // 64):
        pltpu.sync_copy(zrows_v, acc.at[pl.ds(s * (NPAD // NS) + j * 64, 64)])

    rho16 = rho_v[...]

    def mbody(i, carry):
        df = diff_v[pl.ds(i * L, L)]
        dv = dst_v[pl.ds(i * L, L)]
        dst_v[pl.ds(i * L, L)] = jnp.where(df < rho16, dv, DUMMY)
        return carry

    lax.fori_loop(0, EPT // L, mbody, 0)
    plsc.subcore_barrier()

    def chunk(g, carry):
        gb = g * G
        pltpu.async_copy(feat_hbm.at[src_v.at[pl.ds(gb, G)]], rows_v, sem
                         ).wait()

        def stage(j, carry2):
            idx_v[pl.ds(j * L, L)] = dst_v[pl.ds(gb + j * L, L)]
            return carry2

        lax.fori_loop(0, G // L, stage, 0)
        pltpu.sync_copy(rows_v, acc.at[idx_v], add=True)
        return carry

    lax.fori_loop(0, NCHUNK, chunk, 0)
    plsc.subcore_barrier()

    pltpu.async_copy(acc.at[pick_v], out_v, sem).wait()
    pltpu.sync_copy(out_v, neigh_hbm.at[c, pl.ds(s * BPT, BPT)])


def _final_body(ctr_ref, neigh_ref, w_ref, out_ref):
    x = (ctr_ref[0] + ctr_ref[1]) * 0.5 + neigh_ref[0] + neigh_ref[1]
    y = lax.dot_general(x, w_ref[...], (((1,), (1,)), ((), ())),
                        preferred_element_type=jnp.float32)
    out_ref[...] = jnp.maximum(y, 0.0)


def _final(ctr, neigh, weight):
    return pl.pallas_call(
        _final_body,
        out_shape=jax.ShapeDtypeStruct((B, DF), jnp.float32),
    )(ctr, neigh, weight)


def kernel(features, weight, W1, b1, W2, b2, picked_nodes, edge_index):
    d = _dnet(features, W1, b1, W2, b2)
    diffs, ctr = _k2(d, edge_index, picked_nodes, features)
    rho = _rho(diffs)
    (neigh,) = _k4(rho, diffs, edge_index, picked_nodes, features)
    return _final(ctr, neigh, weight)


# speedup vs baseline: 23.3027x; 23.3027x over previous
import functools

import jax
import jax.numpy as jnp
from jax import lax
from jax.experimental import pallas as pl
from jax.experimental.pallas import tpu as pltpu
from jax.experimental.pallas import tpu_sc as plsc

N = 10000
DF = 128
R = 2
B = 1024
E = 160000
TOPK = E // 2

NC, NS, L = 2, 16, 16
EPT = E // NS
BPT = B // NS
G = 80
NCHUNK = EPT // G
NPAD = 10016
DUMMY = N

_mesh = plsc.VectorSubcoreMesh(
    core_axis_name="c", subcore_axis_name="s", num_cores=NC, num_subcores=NS)


def _dnet_body(f_ref, W1_ref, b1_ref, W2_ref, b2_ref, d_ref):
    f = f_ref[...]
    for r in range(R):
        h = jnp.dot(f, W1_ref[r], preferred_element_type=jnp.float32)
        h = jnp.maximum(h + b1_ref[r][None, :], 0.0)
        z = jnp.dot(h, W2_ref[r], preferred_element_type=jnp.float32)
        d_ref[r] = jax.nn.sigmoid(z[:, 1] + b2_ref[r, 1])


def _dnet(features, W1, b1, W2, b2):
    return pl.pallas_call(
        _dnet_body,
        out_shape=jax.ShapeDtypeStruct((R, N), jnp.float32),
    )(features, W1, b1, W2, b2)


@functools.partial(
    pl.kernel,
    out_type=[jax.ShapeDtypeStruct((R * E,), jnp.float32),
              jax.ShapeDtypeStruct((R, B, DF), jnp.float32)],
    mesh=_mesh,
    compiler_params=pltpu.CompilerParams(needs_layout_passes=False),
    scratch_types=[pltpu.VMEM((N,), jnp.float32),
                   pltpu.VMEM((EPT,), jnp.int32),
                   pltpu.VMEM((EPT,), jnp.int32),
                   pltpu.VMEM((EPT,), jnp.float32),
                   pltpu.VMEM((BPT,), jnp.int32),
                   pltpu.VMEM((BPT, DF), jnp.float32),
                   pltpu.SemaphoreType.DMA],
)
def _k2(d_hbm, edge_hbm, picked_hbm, feat_hbm, diffs_hbm, ctr_hbm,
        d_v, src_v, dst_v, diff_v, pick_v, rows_v, sem):
    c = lax.axis_index("c")
    s = lax.axis_index("s")
    base = s * EPT
    pltpu.sync_copy(d_hbm.at[pl.ds(c * N, N)], d_v)
    pltpu.sync_copy(edge_hbm.at[pl.ds(c * 2 * E + base, EPT)], src_v)
    pltpu.sync_copy(edge_hbm.at[pl.ds(c * 2 * E + E + base, EPT)], dst_v)

    def body(i, carry):
        si = src_v[pl.ds(i * L, L)]
        di = dst_v[pl.ds(i * L, L)]
        a = plsc.load_gather(d_v, [si])
        b = plsc.load_gather(d_v, [di])
        diff_v[pl.ds(i * L, L)] = jnp.abs(b - a)
        return carry

    lax.fori_loop(0, EPT // L, body, 0)
    pltpu.sync_copy(diff_v, diffs_hbm.at[pl.ds(c * E + base, EPT)])

    pltpu.sync_copy(picked_hbm.at[pl.ds(c * B + s * BPT, BPT)], pick_v)
    pltpu.async_copy(feat_hbm.at[pick_v], rows_v, sem).wait()
    pltpu.sync_copy(rows_v, ctr_hbm.at[c, pl.ds(s * BPT, BPT)])


def _rho_body(diffs_ref, rho_ref):
    d = diffs_ref[...]
    bits = lax.bitcast_convert_type(d, jnp.int32)

    def step(i, lohi):
        lo, hi = lohi
        mid = (lo + hi + 1) // 2
        cnt = jnp.sum((bits >= mid).astype(jnp.int32), axis=1, keepdims=True)
        take = cnt >= TOPK
        return jnp.where(take, mid, lo), jnp.where(take, hi, mid - 1)

    lo0 = jnp.zeros((R, 1), jnp.int32)
    hi0 = jnp.full((R, 1), 0x3F800000, jnp.int32)
    lo, _ = lax.fori_loop(0, 31, step, (lo0, hi0))
    t = lax.bitcast_convert_type(lo, jnp.float32)
    gt = d > t
    cnt_gt = jnp.sum(gt.astype(jnp.float32), axis=1, keepdims=True)
    sum_gt = jnp.sum(jnp.where(gt, d, 0.0), axis=1, keepdims=True)
    rho = (sum_gt + t * (TOPK - cnt_gt)) * (1.0 / TOPK)
    rho_ref[...] = jnp.broadcast_to(rho, (R, 128))


def _rho(diffs):
    return pl.pallas_call(
        _rho_body,
        out_shape=jax.ShapeDtypeStruct((R, 128), jnp.float32),
    )(diffs)


@functools.partial(
    pl.kernel,
    out_type=[jax.ShapeDtypeStruct((R, B, DF), jnp.float32)],
    mesh=_mesh,
    compiler_params=pltpu.CompilerParams(needs_layout_passes=False),
    scratch_types=[pltpu.VMEM((EPT,), jnp.int32),
                   pltpu.VMEM((EPT,), jnp.int32),
                   pltpu.VMEM((EPT,), jnp.float32),
                   pltpu.VMEM((L,), jnp.float32),
                   pltpu.VMEM((BPT,), jnp.int32),
                   pltpu.VMEM((G,), jnp.int32),
                   pltpu.VMEM((G, DF), jnp.float32),
                   pltpu.VMEM((BPT, DF), jnp.float32),
                   pltpu.VMEM_SHARED((NPAD, DF), jnp.float32),
                   pltpu.SemaphoreType.DMA],
)
def _k4(rho_hbm, diffs_hbm, edge_hbm, picked_hbm, feat_hbm, neigh_hbm,
        src_v, dst_v, diff_v, rho_v, pick_v, idx_v, rows_v,
        out_v, acc, sem):
    c = lax.axis_index("c")
    s = lax.axis_index("s")
    base = s * EPT

    pltpu.sync_copy(edge_hbm.at[pl.ds(c * 2 * E + base, EPT)], src_v)
    pltpu.sync_copy(edge_hbm.at[pl.ds(c * 2 * E + E + base, EPT)], dst_v)
    pltpu.sync_copy(diffs_hbm.at[pl.ds(c * E + base, EPT)], diff_v)
    pltpu.sync_copy(rho_hbm.at[pl.ds(c * 128, L)], rho_v)
    pltpu.sync_copy(picked_hbm.at[pl.ds(c * B + s * BPT, BPT)], pick_v)

    def zb(i, carry):
        def zb2(j, carry2):
            rows_v[i, pl.ds(j * L, L)] = jnp.zeros((L,), jnp.float32)
            return carry2
        return lax.fori_loop(0, DF // L, zb2, carry)
    lax.fori_loop(0, G, zb, 0)
    zbase = s * (NPAD // NS)
    for j in range(7):
        pltpu.sync_copy(rows_v, acc.at[pl.ds(zbase + j * G, G)])
    pltpu.sync_copy(rows_v.at[pl.ds(0, 66)], acc.at[pl.ds(zbase + 560, 66)])

    rho16 = rho_v[...]

    def mbody(i, carry):
        df = diff_v[pl.ds(i * L, L)]
        dv = dst_v[pl.ds(i * L, L)]
        dst_v[pl.ds(i * L, L)] = jnp.where(df < rho16, dv, DUMMY)
        return carry

    lax.fori_loop(0, EPT // L, mbody, 0)
    plsc.subcore_barrier()

    def chunk(g, carry):
        gb = g * G
        pltpu.async_copy(feat_hbm.at[src_v.at[pl.ds(gb, G)]], rows_v, sem
                         ).wait()

        def stage(j, carry2):
            idx_v[pl.ds(j * L, L)] = dst_v[pl.ds(gb + j * L, L)]
            return carry2

        lax.fori_loop(0, G // L, stage, 0)
        pltpu.sync_copy(rows_v, acc.at[idx_v], add=True)
        return carry

    lax.fori_loop(0, NCHUNK, chunk, 0)
    plsc.subcore_barrier()

    pltpu.async_copy(acc.at[pick_v], out_v, sem).wait()
    pltpu.sync_copy(out_v, neigh_hbm.at[c, pl.ds(s * BPT, BPT)])


def _final_body(ctr_ref, neigh_ref, w_ref, out_ref):
    x = (ctr_ref[0] + ctr_ref[1]) * 0.5 + neigh_ref[0] + neigh_ref[1]
    y = lax.dot_general(x, w_ref[...], (((1,), (1,)), ((), ())),
                        preferred_element_type=jnp.float32)
    out_ref[...] = jnp.maximum(y, 0.0)


def _final(ctr, neigh, weight):
    return pl.pallas_call(
        _final_body,
        out_shape=jax.ShapeDtypeStruct((B, DF), jnp.float32),
    )(ctr, neigh, weight)


def kernel(features, weight, W1, b1, W2, b2, picked_nodes, edge_index):
    d = _dnet(features, W1, b1, W2, b2).reshape(R * N)
    edge_flat = edge_index.reshape(R * 2 * E)
    picked_flat = picked_nodes.reshape(R * B)
    diffs, ctr = _k2(d, edge_flat, picked_flat, features)
    rho = _rho(diffs.reshape(R, E)).reshape(R * 128)
    (neigh,) = _k4(rho, diffs, edge_flat, picked_flat, features)
    return _final(ctr, neigh, weight)


# speedup vs baseline: 38.1947x; 1.6391x over previous
import functools

import jax
import jax.numpy as jnp
from jax import lax
from jax.experimental import pallas as pl
from jax.experimental.pallas import tpu as pltpu
from jax.experimental.pallas import tpu_sc as plsc

N = 10000
DF = 128
R = 2
B = 1024
E = 160000
TOPK = E // 2

NC, NS, L = 2, 16, 16
EPT = E // NS
BPT = B // NS
G = 80
NCHUNK = EPT // G
NPAD = 10016
DUMMY = N

_mesh = plsc.VectorSubcoreMesh(
    core_axis_name="c", subcore_axis_name="s", num_cores=NC, num_subcores=NS)


def _dnet_body(f_ref, W1_ref, b1_ref, W2_ref, b2_ref, d_ref):
    f = f_ref[...]
    for r in range(R):
        h = jnp.dot(f, W1_ref[r], preferred_element_type=jnp.float32)
        h = jnp.maximum(h + b1_ref[r][None, :], 0.0)
        z = jnp.dot(h, W2_ref[r], preferred_element_type=jnp.float32)
        d_ref[r] = jax.nn.sigmoid(z[:, 1] + b2_ref[r, 1])


def _dnet(features, W1, b1, W2, b2):
    return pl.pallas_call(
        _dnet_body,
        out_shape=jax.ShapeDtypeStruct((R, N), jnp.float32),
    )(features, W1, b1, W2, b2)


@functools.partial(
    pl.kernel,
    out_type=[jax.ShapeDtypeStruct((R * E,), jnp.float32),
              jax.ShapeDtypeStruct((R, B, DF), jnp.float32)],
    mesh=_mesh,
    compiler_params=pltpu.CompilerParams(needs_layout_passes=False),
    scratch_types=[pltpu.VMEM((N,), jnp.float32),
                   pltpu.VMEM((N,), jnp.float32),
                   pltpu.VMEM((EPT,), jnp.int32),
                   pltpu.VMEM((EPT,), jnp.int32),
                   pltpu.VMEM((EPT,), jnp.float32),
                   pltpu.VMEM((B,), jnp.int32),
                   pltpu.VMEM((BPT, DF), jnp.float32),
                   pltpu.SemaphoreType.DMA],
)
def _k2(d_hbm, edge_hbm, picked_hbm, feat_hbm, diffs_hbm, ctr_hbm,
        d_v, flag_v, src_v, dst_v, diff_v, pickall_v, rows_v, sem):
    c = lax.axis_index("c")
    s = lax.axis_index("s")
    base = s * EPT
    pltpu.sync_copy(d_hbm.at[pl.ds(c * N, N)], d_v)
    pltpu.sync_copy(edge_hbm.at[pl.ds(c * 2 * E + base, EPT)], src_v)
    pltpu.sync_copy(edge_hbm.at[pl.ds(c * 2 * E + E + base, EPT)], dst_v)
    pltpu.sync_copy(picked_hbm.at[pl.ds(c * B, B)], pickall_v)

    def fz(i, carry):
        flag_v[pl.ds(i * L, L)] = jnp.zeros((L,), jnp.float32)
        return carry

    lax.fori_loop(0, N // L, fz, 0)

    def fs(i, carry):
        idx = pickall_v[pl.ds(i * L, L)]
        plsc.store_scatter(flag_v, [idx], jnp.ones((L,), jnp.float32))
        return carry

    lax.fori_loop(0, B // L, fs, 0)

    def body(i, carry):
        si = src_v[pl.ds(i * L, L)]
        di = dst_v[pl.ds(i * L, L)]
        a = plsc.load_gather(d_v, [si])
        b = plsc.load_gather(d_v, [di])
        fl = plsc.load_gather(flag_v, [di])
        ad = jnp.abs(b - a)
        diff_v[pl.ds(i * L, L)] = jnp.where(fl > 0.0, ad, -ad)
        return carry

    lax.fori_loop(0, EPT // L, body, 0)
    pltpu.sync_copy(diff_v, diffs_hbm.at[pl.ds(c * E + base, EPT)])

    pltpu.async_copy(feat_hbm.at[pickall_v.at[pl.ds(s * BPT, BPT)]],
                     rows_v, sem).wait()
    pltpu.sync_copy(rows_v, ctr_hbm.at[c, pl.ds(s * BPT, BPT)])


def _rho_body(diffs_ref, rho_ref):
    d = jnp.abs(diffs_ref[...])
    bits = lax.bitcast_convert_type(d, jnp.int32)

    def step(i, lohi):
        lo, hi = lohi
        mid = (lo + hi + 1) // 2
        cnt = jnp.sum((bits >= mid).astype(jnp.int32), axis=1, keepdims=True)
        take = cnt >= TOPK
        return jnp.where(take, mid, lo), jnp.where(take, hi, mid - 1)

    lo0 = jnp.zeros((R, 1), jnp.int32)
    hi0 = jnp.full((R, 1), 0x3F800000, jnp.int32)
    lo, _ = lax.fori_loop(0, 31, step, (lo0, hi0))
    t = lax.bitcast_convert_type(lo, jnp.float32)
    gt = d > t
    cnt_gt = jnp.sum(gt.astype(jnp.float32), axis=1, keepdims=True)
    sum_gt = jnp.sum(jnp.where(gt, d, 0.0), axis=1, keepdims=True)
    rho = (sum_gt + t * (TOPK - cnt_gt)) * (1.0 / TOPK)
    rho_ref[...] = jnp.broadcast_to(rho, (R, 128))


def _rho(diffs):
    return pl.pallas_call(
        _rho_body,
        out_shape=jax.ShapeDtypeStruct((R, 128), jnp.float32),
    )(diffs)


@functools.partial(
    pl.kernel,
    out_type=[jax.ShapeDtypeStruct((R, B, DF), jnp.float32)],
    mesh=_mesh,
    compiler_params=pltpu.CompilerParams(needs_layout_passes=False),
    scratch_types=[pltpu.VMEM((EPT + 96,), jnp.int32),
                   pltpu.VMEM((EPT + 96,), jnp.int32),
                   pltpu.VMEM((EPT,), jnp.float32),
                   pltpu.VMEM((L,), jnp.float32),
                   pltpu.VMEM((BPT,), jnp.int32),
                   pltpu.VMEM((G,), jnp.int32),
                   pltpu.VMEM((G, DF), jnp.float32),
                   pltpu.VMEM((BPT, DF), jnp.float32),
                   pltpu.VMEM_SHARED((NPAD, DF), jnp.float32),
                   pltpu.SemaphoreType.DMA],
)
def _k4(rho_hbm, diffs_hbm, edge_hbm, picked_hbm, feat_hbm, neigh_hbm,
        src_v, dst_v, diff_v, rho_v, pick_v, idx_v, rows_v,
        out_v, acc, sem):
    c = lax.axis_index("c")
    s = lax.axis_index("s")
    base = s * EPT

    pltpu.sync_copy(edge_hbm.at[pl.ds(c * 2 * E + base, EPT)],
                    src_v.at[pl.ds(0, EPT)])
    pltpu.sync_copy(edge_hbm.at[pl.ds(c * 2 * E + E + base, EPT)],
                    dst_v.at[pl.ds(0, EPT)])
    pltpu.sync_copy(diffs_hbm.at[pl.ds(c * E + base, EPT)], diff_v)
    pltpu.sync_copy(rho_hbm.at[pl.ds(c * 128, L)], rho_v)
    pltpu.sync_copy(picked_hbm.at[pl.ds(c * B + s * BPT, BPT)], pick_v)

    def zb(i, carry):
        def zb2(j, carry2):
            rows_v[i, pl.ds(j * L, L)] = jnp.zeros((L,), jnp.float32)
            return carry2
        return lax.fori_loop(0, DF // L, zb2, carry)
    lax.fori_loop(0, G, zb, 0)
    zbase = s * (NPAD // NS)
    for j in range(7):
        pltpu.sync_copy(rows_v, acc.at[pl.ds(zbase + j * G, G)])
    pltpu.sync_copy(rows_v.at[pl.ds(0, 66)], acc.at[pl.ds(zbase + 560, 66)])

    rho16 = rho_v[...]

    def mbody(i, cnt):
        df = diff_v[pl.ds(i * L, L)]
        sv = src_v[pl.ds(i * L, L)]
        dv = dst_v[pl.ds(i * L, L)]
        keep = jnp.logical_and(
            plsc.bitcast(df, jnp.int32) >= 0, jnp.abs(df) < rho16)
        plsc.store_compressed(src_v.at[pl.ds(cnt, L)], sv, mask=keep)
        plsc.store_compressed(dst_v.at[pl.ds(cnt, L)], dv, mask=keep)
        return cnt + jnp.sum(keep.astype(jnp.int32))

    cnt = lax.fori_loop(0, EPT // L, mbody, 0)

    for j in range(G // L + 1):
        src_v[pl.ds(cnt + j * L, L)] = jnp.zeros((L,), jnp.int32)
        dst_v[pl.ds(cnt + j * L, L)] = jnp.full((L,), DUMMY, jnp.int32)
    plsc.subcore_barrier()

    def chunk(g, carry):
        gb = g * G
        pltpu.async_copy(feat_hbm.at[src_v.at[pl.ds(gb, G)]], rows_v, sem
                         ).wait()

        def stage(j, carry2):
            idx_v[pl.ds(j * L, L)] = dst_v[pl.ds(gb + j * L, L)]
            return carry2

        lax.fori_loop(0, G // L, stage, 0)
        pltpu.sync_copy(rows_v, acc.at[idx_v], add=True)
        return carry

    lax.fori_loop(0, (cnt + G - 1) // G, chunk, 0)
    plsc.subcore_barrier()

    pltpu.async_copy(acc.at[pick_v], out_v, sem).wait()
    pltpu.sync_copy(out_v, neigh_hbm.at[c, pl.ds(s * BPT, BPT)])


def _final_body(ctr_ref, neigh_ref, w_ref, out_ref):
    x = (ctr_ref[0] + ctr_ref[1]) * 0.5 + neigh_ref[0] + neigh_ref[1]
    y = lax.dot_general(x, w_ref[...], (((1,), (1,)), ((), ())),
                        preferred_element_type=jnp.float32)
    out_ref[...] = jnp.maximum(y, 0.0)


def _final(ctr, neigh, weight):
    return pl.pallas_call(
        _final_body,
        out_shape=jax.ShapeDtypeStruct((B, DF), jnp.float32),
    )(ctr, neigh, weight)


def kernel(features, weight, W1, b1, W2, b2, picked_nodes, edge_index):
    d = _dnet(features, W1, b1, W2, b2).reshape(R * N)
    edge_flat = edge_index.reshape(R * 2 * E)
    picked_flat = picked_nodes.reshape(R * B)
    diffs, ctr = _k2(d, edge_flat, picked_flat, features)
    rho = _rho(diffs.reshape(R, E)).reshape(R * 128)
    (neigh,) = _k4(rho, diffs, edge_flat, picked_flat, features)
    return _final(ctr, neigh, weight)


# speedup vs baseline: 45.2612x; 1.1850x over previous
import functools

import jax
import jax.numpy as jnp
from jax import lax
from jax.experimental import pallas as pl
from jax.experimental.pallas import tpu as pltpu
from jax.experimental.pallas import tpu_sc as plsc

N = 10000
DF = 128
R = 2
B = 1024
E = 160000
TOPK = E // 2

NC, NS, L = 2, 16, 16
EPT = E // NS
BPT = B // NS
G = 64
NPAD = 10016
DUMMY = N

_mesh = plsc.VectorSubcoreMesh(
    core_axis_name="c", subcore_axis_name="s", num_cores=NC, num_subcores=NS)


def _dnet_body(f_ref, W1_ref, b1_ref, W2_ref, b2_ref, d_ref):
    f = f_ref[...]
    for r in range(R):
        h = jnp.dot(f, W1_ref[r], preferred_element_type=jnp.float32)
        h = jnp.maximum(h + b1_ref[r][None, :], 0.0)
        z = jnp.dot(h, W2_ref[r], preferred_element_type=jnp.float32)
        d_ref[r] = jax.nn.sigmoid(z[:, 1] + b2_ref[r, 1])


def _dnet(features, W1, b1, W2, b2):
    return pl.pallas_call(
        _dnet_body,
        out_shape=jax.ShapeDtypeStruct((R, N), jnp.float32),
    )(features, W1, b1, W2, b2)


@functools.partial(
    pl.kernel,
    out_type=[jax.ShapeDtypeStruct((R * E,), jnp.float32),
              jax.ShapeDtypeStruct((R, B, DF), jnp.float32)],
    mesh=_mesh,
    compiler_params=pltpu.CompilerParams(needs_layout_passes=False),
    scratch_types=[pltpu.VMEM((N,), jnp.float32),
                   pltpu.VMEM((N,), jnp.float32),
                   pltpu.VMEM((EPT,), jnp.int32),
                   pltpu.VMEM((EPT,), jnp.int32),
                   pltpu.VMEM((EPT,), jnp.float32),
                   pltpu.VMEM((B,), jnp.int32),
                   pltpu.VMEM((BPT, DF), jnp.float32),
                   pltpu.SemaphoreType.DMA],
)
def _k2(d_hbm, edge_hbm, picked_hbm, feat_hbm, diffs_hbm, ctr_hbm,
        d_v, flag_v, src_v, dst_v, diff_v, pickall_v, rows_v, sem):
    c = lax.axis_index("c")
    s = lax.axis_index("s")
    base = s * EPT
    pltpu.sync_copy(d_hbm.at[pl.ds(c * N, N)], d_v)
    pltpu.sync_copy(edge_hbm.at[pl.ds(c * 2 * E + base, EPT)], src_v)
    pltpu.sync_copy(edge_hbm.at[pl.ds(c * 2 * E + E + base, EPT)], dst_v)
    pltpu.sync_copy(picked_hbm.at[pl.ds(c * B, B)], pickall_v)

    def fz(i, carry):
        flag_v[pl.ds(i * L, L)] = jnp.zeros((L,), jnp.float32)
        return carry

    lax.fori_loop(0, N // L, fz, 0)

    def fs(i, carry):
        idx = pickall_v[pl.ds(i * L, L)]
        plsc.store_scatter(flag_v, [idx], jnp.ones((L,), jnp.float32))
        return carry

    lax.fori_loop(0, B // L, fs, 0)

    def body(i, carry):
        si = src_v[pl.ds(i * L, L)]
        di = dst_v[pl.ds(i * L, L)]
        a = plsc.load_gather(d_v, [si])
        b = plsc.load_gather(d_v, [di])
        fl = plsc.load_gather(flag_v, [di])
        ad = jnp.abs(b - a)
        diff_v[pl.ds(i * L, L)] = jnp.where(fl > 0.0, ad, -ad)
        return carry

    lax.fori_loop(0, EPT // L, body, 0)
    pltpu.sync_copy(diff_v, diffs_hbm.at[pl.ds(c * E + base, EPT)])

    pltpu.async_copy(feat_hbm.at[pickall_v.at[pl.ds(s * BPT, BPT)]],
                     rows_v, sem).wait()
    pltpu.sync_copy(rows_v, ctr_hbm.at[c, pl.ds(s * BPT, BPT)])


def _rho_body(diffs_ref, rho_ref):
    d = jnp.abs(diffs_ref[...])
    bits = lax.bitcast_convert_type(d, jnp.int32)

    def step(i, lohi):
        lo, hi = lohi
        mid = (lo + hi + 1) // 2
        cnt = jnp.sum((bits >= mid).astype(jnp.int32), axis=1, keepdims=True)
        take = cnt >= TOPK
        return jnp.where(take, mid, lo), jnp.where(take, hi, mid - 1)

    lo0 = jnp.zeros((R, 1), jnp.int32)
    hi0 = jnp.full((R, 1), 0x3F800000, jnp.int32)
    lo, _ = lax.fori_loop(0, 31, step, (lo0, hi0))
    t = lax.bitcast_convert_type(lo, jnp.float32)
    gt = d > t
    cnt_gt = jnp.sum(gt.astype(jnp.float32), axis=1, keepdims=True)
    sum_gt = jnp.sum(jnp.where(gt, d, 0.0), axis=1, keepdims=True)
    rho = (sum_gt + t * (TOPK - cnt_gt)) * (1.0 / TOPK)
    rho_ref[...] = jnp.broadcast_to(rho, (R, 128))


def _rho(diffs):
    return pl.pallas_call(
        _rho_body,
        out_shape=jax.ShapeDtypeStruct((R, 128), jnp.float32),
    )(diffs)


@functools.partial(
    pl.kernel,
    out_type=[jax.ShapeDtypeStruct((R, B, DF), jnp.float32)],
    mesh=_mesh,
    compiler_params=pltpu.CompilerParams(needs_layout_passes=False),
    scratch_types=[pltpu.VMEM((EPT + 96,), jnp.int32),
                   pltpu.VMEM((EPT + 96,), jnp.int32),
                   pltpu.VMEM((EPT,), jnp.float32),
                   pltpu.VMEM((L,), jnp.float32),
                   pltpu.VMEM((BPT,), jnp.int32),
                   pltpu.VMEM((G,), jnp.int32),
                   pltpu.VMEM((G,), jnp.int32),
                   pltpu.VMEM((G, DF), jnp.float32),
                   pltpu.VMEM((G, DF), jnp.float32),
                   pltpu.VMEM_SHARED((NPAD, DF), jnp.float32),
                   pltpu.SemaphoreType.DMA,
                   pltpu.SemaphoreType.DMA],
)
def _k4(rho_hbm, diffs_hbm, edge_hbm, picked_hbm, feat_hbm, neigh_hbm,
        src_v, dst_v, diff_v, rho_v, pick_v, idx0_v, idx1_v,
        rows0_v, rows1_v, acc, sem0, sem1):
    c = lax.axis_index("c")
    s = lax.axis_index("s")
    base = s * EPT

    pltpu.sync_copy(edge_hbm.at[pl.ds(c * 2 * E + base, EPT)],
                    src_v.at[pl.ds(0, EPT)])
    pltpu.sync_copy(edge_hbm.at[pl.ds(c * 2 * E + E + base, EPT)],
                    dst_v.at[pl.ds(0, EPT)])
    pltpu.sync_copy(diffs_hbm.at[pl.ds(c * E + base, EPT)], diff_v)
    pltpu.sync_copy(rho_hbm.at[pl.ds(c * 128, L)], rho_v)
    pltpu.sync_copy(picked_hbm.at[pl.ds(c * B + s * BPT, BPT)], pick_v)

    def zb(i, carry):
        def zb2(j, carry2):
            rows0_v[i, pl.ds(j * L, L)] = jnp.zeros((L,), jnp.float32)
            return carry2
        return lax.fori_loop(0, DF // L, zb2, carry)
    lax.fori_loop(0, G, zb, 0)
    zbase = s * (NPAD // NS)
    for j in range(9):
        pltpu.sync_copy(rows0_v, acc.at[pl.ds(zbase + j * G, G)])
    pltpu.sync_copy(rows0_v.at[pl.ds(0, 50)], acc.at[pl.ds(zbase + 576, 50)])

    rho16 = rho_v[...]

    def judge(off):
        df = diff_v[pl.ds(off, L)]
        sv = src_v[pl.ds(off, L)]
        dv = dst_v[pl.ds(off, L)]
        keep = jnp.logical_and(
            plsc.bitcast(df, jnp.int32) >= 0, jnp.abs(df) < rho16)
        return sv, dv, keep, jnp.sum(keep.astype(jnp.int32))

    def emit(cnt, rec):
        sv, dv, keep, ksum = rec
        plsc.store_compressed(src_v.at[pl.ds(cnt, L)], sv, mask=keep)
        plsc.store_compressed(dst_v.at[pl.ds(cnt, L)], dv, mask=keep)
        return cnt + ksum

    UNROLL = 4

    def mbody(i, cnt):
        recs = [judge((i * UNROLL + u) * L) for u in range(UNROLL)]
        for rec in recs:
            cnt = emit(cnt, rec)
        return cnt

    NB = EPT // L
    cnt = lax.fori_loop(0, NB // UNROLL, mbody, 0)
    for u in range(NB - (NB // UNROLL) * UNROLL):
        cnt = emit(cnt, judge((NB - NB % UNROLL + u) * L))

    for j in range(G // L + 1):
        src_v[pl.ds(cnt + j * L, L)] = jnp.zeros((L,), jnp.int32)
        dst_v[pl.ds(cnt + j * L, L)] = jnp.full((L,), DUMMY, jnp.int32)
    plsc.subcore_barrier()

    nchunks = (cnt + G - 1) // G

    def gather(g, buf, sem):
        return pltpu.async_copy(
            feat_hbm.at[src_v.at[pl.ds(g * G, G)]], buf, sem)

    def drain(g, buf, idx, sem):
        pltpu.make_async_copy(
            feat_hbm.at[src_v.at[pl.ds(g * G, G)]], buf, sem).wait()

        def stage(j, carry2):
            idx[pl.ds(j * L, L)] = dst_v[pl.ds(g * G + j * L, L)]
            return carry2

        lax.fori_loop(0, G // L, stage, 0)
        pltpu.sync_copy(buf, acc.at[idx], add=True)

    @pl.when(nchunks > 0)
    def _():
        gather(0, rows0_v, sem0)

    def pair(p, carry):
        even = 2 * p
        odd = even + 1

        @pl.when(odd < nchunks)
        def _():
            gather(odd, rows1_v, sem1)

        drain(even, rows0_v, idx0_v, sem0)

        @pl.when(odd < nchunks)
        def _():
            @pl.when(odd + 1 < nchunks)
            def _():
                gather(odd + 1, rows0_v, sem0)

            drain(odd, rows1_v, idx1_v, sem1)

        return carry

    lax.fori_loop(0, (nchunks + 1) // 2, pair, 0)
    plsc.subcore_barrier()

    pltpu.async_copy(acc.at[pick_v], rows0_v, sem0).wait()
    pltpu.sync_copy(rows0_v, neigh_hbm.at[c, pl.ds(s * BPT, BPT)])


def _final_body(ctr_ref, neigh_ref, w_ref, out_ref):
    x = (ctr_ref[0] + ctr_ref[1]) * 0.5 + neigh_ref[0] + neigh_ref[1]
    y = lax.dot_general(x, w_ref[...], (((1,), (1,)), ((), ())),
                        preferred_element_type=jnp.float32)
    out_ref[...] = jnp.maximum(y, 0.0)


def _final(ctr, neigh, weight):
    return pl.pallas_call(
        _final_body,
        out_shape=jax.ShapeDtypeStruct((B, DF), jnp.float32),
    )(ctr, neigh, weight)


def kernel(features, weight, W1, b1, W2, b2, picked_nodes, edge_index):
    d = _dnet(features, W1, b1, W2, b2).reshape(R * N)
    edge_flat = edge_index.reshape(R * 2 * E)
    picked_flat = picked_nodes.reshape(R * B)
    diffs, ctr = _k2(d, edge_flat, picked_flat, features)
    rho = _rho(diffs.reshape(R, E)).reshape(R * 128)
    (neigh,) = _k4(rho, diffs, edge_flat, picked_flat, features)
    return _final(ctr, neigh, weight)


# speedup vs baseline: 46.6654x; 1.0310x over previous
import functools

import jax
import jax.numpy as jnp
from jax import lax
from jax.experimental import pallas as pl
from jax.experimental.pallas import tpu as pltpu
from jax.experimental.pallas import tpu_sc as plsc

N = 10000
DF = 128
R = 2
B = 1024
E = 160000
TOPK = E // 2

NC, NS, L = 2, 16, 16
EPT = E // NS
BPT = B // NS
G = 64
NPAD = 10016
DUMMY = N

_mesh = plsc.VectorSubcoreMesh(
    core_axis_name="c", subcore_axis_name="s", num_cores=NC, num_subcores=NS)


def _dnet_body(f_ref, W1_ref, b1_ref, W2_ref, b2_ref, d_ref):
    f = f_ref[...]
    for r in range(R):
        h = jnp.dot(f, W1_ref[r], preferred_element_type=jnp.float32)
        h = jnp.maximum(h + b1_ref[r][None, :], 0.0)
        z = jnp.dot(h, W2_ref[r], preferred_element_type=jnp.float32)
        d_ref[r] = jax.nn.sigmoid(z[:, 1] + b2_ref[r, 1])


def _dnet(features, W1, b1, W2, b2):
    return pl.pallas_call(
        _dnet_body,
        out_shape=jax.ShapeDtypeStruct((R, N), jnp.float32),
    )(features, W1, b1, W2, b2)


@functools.partial(
    pl.kernel,
    out_type=[jax.ShapeDtypeStruct((R * E,), jnp.float32),
              jax.ShapeDtypeStruct((R, B, DF), jnp.float32)],
    mesh=_mesh,
    compiler_params=pltpu.CompilerParams(needs_layout_passes=False),
    scratch_types=[pltpu.VMEM((N,), jnp.float32),
                   pltpu.VMEM((N,), jnp.float32),
                   pltpu.VMEM((EPT,), jnp.int32),
                   pltpu.VMEM((EPT,), jnp.int32),
                   pltpu.VMEM((EPT,), jnp.float32),
                   pltpu.VMEM((B,), jnp.int32),
                   pltpu.VMEM((BPT, DF), jnp.float32),
                   pltpu.SemaphoreType.DMA],
)
def _k2(d_hbm, edge_hbm, picked_hbm, feat_hbm, diffs_hbm, ctr_hbm,
        d_v, flag_v, src_v, dst_v, diff_v, pickall_v, rows_v, sem):
    c = lax.axis_index("c")
    s = lax.axis_index("s")
    base = s * EPT
    pltpu.async_copy(d_hbm.at[pl.ds(c * N, N)], d_v, sem)
    pltpu.async_copy(edge_hbm.at[pl.ds(c * 2 * E + base, EPT)], src_v, sem)
    pltpu.async_copy(edge_hbm.at[pl.ds(c * 2 * E + E + base, EPT)], dst_v,
                     sem)
    pltpu.async_copy(picked_hbm.at[pl.ds(c * B, B)], pickall_v, sem)
    pltpu.make_async_copy(d_hbm.at[pl.ds(c * N, N)], d_v, sem).wait()
    pltpu.make_async_copy(edge_hbm.at[pl.ds(c * 2 * E + base, EPT)], src_v,
                          sem).wait()
    pltpu.make_async_copy(edge_hbm.at[pl.ds(c * 2 * E + E + base, EPT)],
                          dst_v, sem).wait()
    pltpu.make_async_copy(picked_hbm.at[pl.ds(c * B, B)], pickall_v,
                          sem).wait()

    def fz(i, carry):
        flag_v[pl.ds(i * L, L)] = jnp.zeros((L,), jnp.float32)
        return carry

    lax.fori_loop(0, N // L, fz, 0)

    def fs(i, carry):
        idx = pickall_v[pl.ds(i * L, L)]
        plsc.store_scatter(flag_v, [idx], jnp.ones((L,), jnp.float32))
        return carry

    lax.fori_loop(0, B // L, fs, 0)

    def body(i, carry):
        for u in range(4):
            off = (i * 4 + u) * L
            si = src_v[pl.ds(off, L)]
            di = dst_v[pl.ds(off, L)]
            a = plsc.load_gather(d_v, [si])
            b = plsc.load_gather(d_v, [di])
            fl = plsc.load_gather(flag_v, [di])
            ad = jnp.abs(b - a)
            diff_v[pl.ds(off, L)] = jnp.where(fl > 0.0, ad, -ad)
        return carry

    lax.fori_loop(0, EPT // L // 4, body, 0)
    for u in range(EPT // L - (EPT // L // 4) * 4):
        i = EPT // L - EPT // L % 4 + u
        si = src_v[pl.ds(i * L, L)]
        di = dst_v[pl.ds(i * L, L)]
        a = plsc.load_gather(d_v, [si])
        b = plsc.load_gather(d_v, [di])
        fl = plsc.load_gather(flag_v, [di])
        ad = jnp.abs(b - a)
        diff_v[pl.ds(i * L, L)] = jnp.where(fl > 0.0, ad, -ad)
    pltpu.sync_copy(diff_v, diffs_hbm.at[pl.ds(c * E + base, EPT)])

    pltpu.async_copy(feat_hbm.at[pickall_v.at[pl.ds(s * BPT, BPT)]],
                     rows_v, sem).wait()
    pltpu.sync_copy(rows_v, ctr_hbm.at[c, pl.ds(s * BPT, BPT)])


def _rho_body(diffs_ref, rho_ref):
    d = jnp.abs(diffs_ref[...])
    bits = lax.bitcast_convert_type(d, jnp.int32)

    def step(i, lohi):
        lo, hi = lohi
        mid = (lo + hi + 1) // 2
        cnt = jnp.sum((bits >= mid).astype(jnp.int32), axis=1, keepdims=True)
        take = cnt >= TOPK
        return jnp.where(take, mid, lo), jnp.where(take, hi, mid - 1)

    lo0 = jnp.zeros((R, 1), jnp.int32)
    hi0 = jnp.full((R, 1), 0x3F800000, jnp.int32)
    lo, _ = lax.fori_loop(0, 31, step, (lo0, hi0))
    t = lax.bitcast_convert_type(lo, jnp.float32)
    gt = d > t
    cnt_gt = jnp.sum(gt.astype(jnp.float32), axis=1, keepdims=True)
    sum_gt = jnp.sum(jnp.where(gt, d, 0.0), axis=1, keepdims=True)
    rho = (sum_gt + t * (TOPK - cnt_gt)) * (1.0 / TOPK)
    rho_ref[...] = jnp.broadcast_to(rho, (R, 128))


def _rho(diffs):
    return pl.pallas_call(
        _rho_body,
        out_shape=jax.ShapeDtypeStruct((R, 128), jnp.float32),
    )(diffs)


@functools.partial(
    pl.kernel,
    out_type=[jax.ShapeDtypeStruct((R, B, DF), jnp.float32)],
    mesh=_mesh,
    compiler_params=pltpu.CompilerParams(needs_layout_passes=False),
    scratch_types=[pltpu.VMEM((EPT + 96,), jnp.int32),
                   pltpu.VMEM((EPT + 96,), jnp.int32),
                   pltpu.VMEM((EPT,), jnp.float32),
                   pltpu.VMEM((L,), jnp.float32),
                   pltpu.VMEM((BPT,), jnp.int32),
                   pltpu.VMEM((G,), jnp.int32),
                   pltpu.VMEM((G,), jnp.int32),
                   pltpu.VMEM((G, DF), jnp.float32),
                   pltpu.VMEM((G, DF), jnp.float32),
                   pltpu.VMEM_SHARED((NPAD, DF), jnp.float32),
                   pltpu.SemaphoreType.DMA,
                   pltpu.SemaphoreType.DMA],
)
def _k4(rho_hbm, diffs_hbm, edge_hbm, picked_hbm, feat_hbm, neigh_hbm,
        src_v, dst_v, diff_v, rho_v, pick_v, idx0_v, idx1_v,
        rows0_v, rows1_v, acc, sem0, sem1):
    c = lax.axis_index("c")
    s = lax.axis_index("s")
    base = s * EPT

    pltpu.async_copy(edge_hbm.at[pl.ds(c * 2 * E + base, EPT)],
                     src_v.at[pl.ds(0, EPT)], sem0)
    pltpu.async_copy(edge_hbm.at[pl.ds(c * 2 * E + E + base, EPT)],
                     dst_v.at[pl.ds(0, EPT)], sem0)
    pltpu.async_copy(diffs_hbm.at[pl.ds(c * E + base, EPT)], diff_v, sem0)
    pltpu.async_copy(rho_hbm.at[pl.ds(c * 128, L)], rho_v, sem0)
    pltpu.async_copy(picked_hbm.at[pl.ds(c * B + s * BPT, BPT)], pick_v, sem0)
    pltpu.make_async_copy(edge_hbm.at[pl.ds(c * 2 * E + base, EPT)],
                          src_v.at[pl.ds(0, EPT)], sem0).wait()
    pltpu.make_async_copy(edge_hbm.at[pl.ds(c * 2 * E + E + base, EPT)],
                          dst_v.at[pl.ds(0, EPT)], sem0).wait()
    pltpu.make_async_copy(diffs_hbm.at[pl.ds(c * E + base, EPT)],
                          diff_v, sem0).wait()
    pltpu.make_async_copy(rho_hbm.at[pl.ds(c * 128, L)], rho_v, sem0).wait()
    pltpu.make_async_copy(picked_hbm.at[pl.ds(c * B + s * BPT, BPT)],
                          pick_v, sem0).wait()

    def zb(i, carry):
        def zb2(j, carry2):
            rows0_v[i, pl.ds(j * L, L)] = jnp.zeros((L,), jnp.float32)
            return carry2
        return lax.fori_loop(0, DF // L, zb2, carry)
    lax.fori_loop(0, G, zb, 0)
    zbase = s * (NPAD // NS)
    for j in range(9):
        pltpu.async_copy(rows0_v, acc.at[pl.ds(zbase + j * G, G)], sem1)
    pltpu.async_copy(rows0_v.at[pl.ds(0, 50)], acc.at[pl.ds(zbase + 576, 50)],
                     sem1)
    for j in range(9):
        pltpu.make_async_copy(rows0_v, acc.at[pl.ds(zbase + j * G, G)],
                              sem1).wait()
    pltpu.make_async_copy(rows0_v.at[pl.ds(0, 50)],
                          acc.at[pl.ds(zbase + 576, 50)], sem1).wait()

    rho16 = rho_v[...]

    def judge(off):
        df = diff_v[pl.ds(off, L)]
        sv = src_v[pl.ds(off, L)]
        dv = dst_v[pl.ds(off, L)]
        keep = jnp.logical_and(
            plsc.bitcast(df, jnp.int32) >= 0, jnp.abs(df) < rho16)
        return sv, dv, keep, jnp.sum(keep.astype(jnp.int32))

    def emit(cnt, rec):
        sv, dv, keep, ksum = rec
        plsc.store_compressed(src_v.at[pl.ds(cnt, L)], sv, mask=keep)
        plsc.store_compressed(dst_v.at[pl.ds(cnt, L)], dv, mask=keep)
        return cnt + ksum

    UNROLL = 4

    def mbody(i, cnt):
        recs = [judge((i * UNROLL + u) * L) for u in range(UNROLL)]
        for rec in recs:
            cnt = emit(cnt, rec)
        return cnt

    NB = EPT // L
    cnt = lax.fori_loop(0, NB // UNROLL, mbody, 0)
    for u in range(NB - (NB // UNROLL) * UNROLL):
        cnt = emit(cnt, judge((NB - NB % UNROLL + u) * L))

    for j in range(G // L + 1):
        src_v[pl.ds(cnt + j * L, L)] = jnp.zeros((L,), jnp.int32)
        dst_v[pl.ds(cnt + j * L, L)] = jnp.full((L,), DUMMY, jnp.int32)
    plsc.subcore_barrier()

    nchunks = (cnt + G - 1) // G

    def gather(g, buf, sem):
        return pltpu.async_copy(
            feat_hbm.at[src_v.at[pl.ds(g * G, G)]], buf, sem)

    def drain(g, buf, idx, sem):
        pltpu.make_async_copy(
            feat_hbm.at[src_v.at[pl.ds(g * G, G)]], buf, sem).wait()

        def stage(j, carry2):
            idx[pl.ds(j * L, L)] = dst_v[pl.ds(g * G + j * L, L)]
            return carry2

        lax.fori_loop(0, G // L, stage, 0)
        pltpu.sync_copy(buf, acc.at[idx], add=True)

    @pl.when(nchunks > 0)
    def _():
        gather(0, rows0_v, sem0)

    def pair(p, carry):
        even = 2 * p
        odd = even + 1

        @pl.when(odd < nchunks)
        def _():
            gather(odd, rows1_v, sem1)

        drain(even, rows0_v, idx0_v, sem0)

        @pl.when(odd < nchunks)
        def _():
            @pl.when(odd + 1 < nchunks)
            def _():
                gather(odd + 1, rows0_v, sem0)

            drain(odd, rows1_v, idx1_v, sem1)

        return carry

    lax.fori_loop(0, (nchunks + 1) // 2, pair, 0)
    plsc.subcore_barrier()

    pltpu.async_copy(acc.at[pick_v], rows0_v, sem0).wait()
    pltpu.sync_copy(rows0_v, neigh_hbm.at[c, pl.ds(s * BPT, BPT)])


def _final_body(ctr_ref, neigh_ref, w_ref, out_ref):
    x = (ctr_ref[0] + ctr_ref[1]) * 0.5 + neigh_ref[0] + neigh_ref[1]
    y = lax.dot_general(x, w_ref[...], (((1,), (1,)), ((), ())),
                        preferred_element_type=jnp.float32)
    out_ref[...] = jnp.maximum(y, 0.0)


def _final(ctr, neigh, weight):
    return pl.pallas_call(
        _final_body,
        out_shape=jax.ShapeDtypeStruct((B, DF), jnp.float32),
    )(ctr, neigh, weight)


def kernel(features, weight, W1, b1, W2, b2, picked_nodes, edge_index):
    d = _dnet(features, W1, b1, W2, b2).reshape(R * N)
    edge_flat = edge_index.reshape(R * 2 * E)
    picked_flat = picked_nodes.reshape(R * B)
    diffs, ctr = _k2(d, edge_flat, picked_flat, features)
    rho = _rho(diffs.reshape(R, E)).reshape(R * 128)
    (neigh,) = _k4(rho, diffs, edge_flat, picked_flat, features)
    return _final(ctr, neigh, weight)


# speedup vs baseline: 49.2693x; 1.0558x over previous
import functools

import jax
import jax.numpy as jnp
from jax import lax
from jax.experimental import pallas as pl
from jax.experimental.pallas import tpu as pltpu
from jax.experimental.pallas import tpu_sc as plsc

N = 10000
DF = 128
R = 2
B = 1024
E = 160000
TOPK = E // 2

NC, NS, L = 2, 16, 16
EPT = E // NS
BPT = B // NS
G = 64
NPAD = 10016
DUMMY = N

_mesh = plsc.VectorSubcoreMesh(
    core_axis_name="c", subcore_axis_name="s", num_cores=NC, num_subcores=NS)


def _dnet_body(f_ref, W1_ref, b1_ref, W2_ref, b2_ref, d_ref):
    f = f_ref[...]
    for r in range(R):
        h = jnp.dot(f, W1_ref[r], preferred_element_type=jnp.float32)
        h = jnp.maximum(h + b1_ref[r][None, :], 0.0)
        zt = lax.dot_general(W2_ref[r, :, 1][None, :], h,
                             (((1,), (1,)), ((), ())),
                             preferred_element_type=jnp.float32)
        d_ref[r] = jax.nn.sigmoid(zt[0] + b2_ref[r, 1])


def _dnet(features, W1, b1, W2, b2):
    return pl.pallas_call(
        _dnet_body,
        out_shape=jax.ShapeDtypeStruct((R, N), jnp.float32),
    )(features, W1, b1, W2, b2)


@functools.partial(
    pl.kernel,
    out_type=[jax.ShapeDtypeStruct((R * E,), jnp.float32),
              jax.ShapeDtypeStruct((R, B, DF), jnp.float32)],
    mesh=_mesh,
    compiler_params=pltpu.CompilerParams(needs_layout_passes=False),
    scratch_types=[pltpu.VMEM((N,), jnp.float32),
                   pltpu.VMEM((N,), jnp.float32),
                   pltpu.VMEM((EPT,), jnp.int32),
                   pltpu.VMEM((EPT,), jnp.int32),
                   pltpu.VMEM((EPT,), jnp.float32),
                   pltpu.VMEM((B,), jnp.int32),
                   pltpu.VMEM((BPT, DF), jnp.float32),
                   pltpu.SemaphoreType.DMA],
)
def _k2(d_hbm, edge_hbm, picked_hbm, feat_hbm, diffs_hbm, ctr_hbm,
        d_v, flag_v, src_v, dst_v, diff_v, pickall_v, rows_v, sem):
    c = lax.axis_index("c")
    s = lax.axis_index("s")
    base = s * EPT
    pltpu.async_copy(d_hbm.at[pl.ds(c * N, N)], d_v, sem)
    pltpu.async_copy(edge_hbm.at[pl.ds(c * 2 * E + base, EPT)], src_v, sem)
    pltpu.async_copy(edge_hbm.at[pl.ds(c * 2 * E + E + base, EPT)], dst_v,
                     sem)
    pltpu.async_copy(picked_hbm.at[pl.ds(c * B, B)], pickall_v, sem)
    pltpu.make_async_copy(d_hbm.at[pl.ds(c * N, N)], d_v, sem).wait()
    pltpu.make_async_copy(edge_hbm.at[pl.ds(c * 2 * E + base, EPT)], src_v,
                          sem).wait()
    pltpu.make_async_copy(edge_hbm.at[pl.ds(c * 2 * E + E + base, EPT)],
                          dst_v, sem).wait()
    pltpu.make_async_copy(picked_hbm.at[pl.ds(c * B, B)], pickall_v,
                          sem).wait()

    def fz(i, carry):
        flag_v[pl.ds(i * L, L)] = jnp.zeros((L,), jnp.float32)
        return carry

    lax.fori_loop(0, N // L, fz, 0)

    def fs(i, carry):
        idx = pickall_v[pl.ds(i * L, L)]
        plsc.store_scatter(flag_v, [idx], jnp.ones((L,), jnp.float32))
        return carry

    lax.fori_loop(0, B // L, fs, 0)

    def body(i, carry):
        for u in range(4):
            off = (i * 4 + u) * L
            si = src_v[pl.ds(off, L)]
            di = dst_v[pl.ds(off, L)]
            a = plsc.load_gather(d_v, [si])
            b = plsc.load_gather(d_v, [di])
            fl = plsc.load_gather(flag_v, [di])
            ad = jnp.abs(b - a)
            diff_v[pl.ds(off, L)] = jnp.where(fl > 0.0, ad, -ad)
        return carry

    lax.fori_loop(0, EPT // L // 4, body, 0)
    for u in range(EPT // L - (EPT // L // 4) * 4):
        i = EPT // L - EPT // L % 4 + u
        si = src_v[pl.ds(i * L, L)]
        di = dst_v[pl.ds(i * L, L)]
        a = plsc.load_gather(d_v, [si])
        b = plsc.load_gather(d_v, [di])
        fl = plsc.load_gather(flag_v, [di])
        ad = jnp.abs(b - a)
        diff_v[pl.ds(i * L, L)] = jnp.where(fl > 0.0, ad, -ad)
    pltpu.sync_copy(diff_v, diffs_hbm.at[pl.ds(c * E + base, EPT)])

    pltpu.async_copy(feat_hbm.at[pickall_v.at[pl.ds(s * BPT, BPT)]],
                     rows_v, sem).wait()
    pltpu.sync_copy(rows_v, ctr_hbm.at[c, pl.ds(s * BPT, BPT)])


def _rho_body(diffs_ref, rho_ref):
    d = jnp.abs(diffs_ref[...])
    bits = lax.bitcast_convert_type(d, jnp.int32)

    def step(i, lohi):
        lo, hi = lohi
        mid = (lo + hi + 1) // 2
        cnt = jnp.sum((bits >= mid).astype(jnp.int32), axis=1, keepdims=True)
        take = cnt >= TOPK
        return jnp.where(take, mid, lo), jnp.where(take, hi, mid - 1)

    lo0 = jnp.zeros((R, 1), jnp.int32)
    hi0 = jnp.full((R, 1), 0x3F800000, jnp.int32)
    lo, _ = lax.fori_loop(0, 31, step, (lo0, hi0))
    t = lax.bitcast_convert_type(lo, jnp.float32)
    gt = d > t
    cnt_gt = jnp.sum(gt.astype(jnp.float32), axis=1, keepdims=True)
    sum_gt = jnp.sum(jnp.where(gt, d, 0.0), axis=1, keepdims=True)
    rho = (sum_gt + t * (TOPK - cnt_gt)) * (1.0 / TOPK)
    rho_ref[...] = jnp.broadcast_to(rho, (R, 128))


def _rho(diffs):
    return pl.pallas_call(
        _rho_body,
        out_shape=jax.ShapeDtypeStruct((R, 128), jnp.float32),
    )(diffs)


@functools.partial(
    pl.kernel,
    out_type=[jax.ShapeDtypeStruct((R, B, DF), jnp.float32)],
    mesh=_mesh,
    compiler_params=pltpu.CompilerParams(needs_layout_passes=False),
    scratch_types=[pltpu.VMEM((EPT + 96,), jnp.int32),
                   pltpu.VMEM((EPT + 96,), jnp.int32),
                   pltpu.VMEM((EPT,), jnp.float32),
                   pltpu.VMEM((L,), jnp.float32),
                   pltpu.VMEM((BPT,), jnp.int32),
                   pltpu.VMEM((G,), jnp.int32),
                   pltpu.VMEM((G,), jnp.int32),
                   pltpu.VMEM((G, DF), jnp.float32),
                   pltpu.VMEM((G, DF), jnp.float32),
                   pltpu.VMEM_SHARED((NPAD, DF), jnp.float32),
                   pltpu.SemaphoreType.DMA,
                   pltpu.SemaphoreType.DMA],
)
def _k4(rho_hbm, diffs_hbm, edge_hbm, picked_hbm, feat_hbm, neigh_hbm,
        src_v, dst_v, diff_v, rho_v, pick_v, idx0_v, idx1_v,
        rows0_v, rows1_v, acc, sem0, sem1):
    c = lax.axis_index("c")
    s = lax.axis_index("s")
    base = s * EPT

    pltpu.async_copy(edge_hbm.at[pl.ds(c * 2 * E + base, EPT)],
                     src_v.at[pl.ds(0, EPT)], sem0)
    pltpu.async_copy(edge_hbm.at[pl.ds(c * 2 * E + E + base, EPT)],
                     dst_v.at[pl.ds(0, EPT)], sem0)
    pltpu.async_copy(diffs_hbm.at[pl.ds(c * E + base, EPT)], diff_v, sem0)
    pltpu.async_copy(rho_hbm.at[pl.ds(c * 128, L)], rho_v, sem0)
    pltpu.async_copy(picked_hbm.at[pl.ds(c * B + s * BPT, BPT)], pick_v, sem0)
    pltpu.make_async_copy(edge_hbm.at[pl.ds(c * 2 * E + base, EPT)],
                          src_v.at[pl.ds(0, EPT)], sem0).wait()
    pltpu.make_async_copy(edge_hbm.at[pl.ds(c * 2 * E + E + base, EPT)],
                          dst_v.at[pl.ds(0, EPT)], sem0).wait()
    pltpu.make_async_copy(diffs_hbm.at[pl.ds(c * E + base, EPT)],
                          diff_v, sem0).wait()
    pltpu.make_async_copy(rho_hbm.at[pl.ds(c * 128, L)], rho_v, sem0).wait()
    pltpu.make_async_copy(picked_hbm.at[pl.ds(c * B + s * BPT, BPT)],
                          pick_v, sem0).wait()

    def zb(i, carry):
        def zb2(j, carry2):
            rows0_v[i, pl.ds(j * L, L)] = jnp.zeros((L,), jnp.float32)
            return carry2
        return lax.fori_loop(0, DF // L, zb2, carry)
    lax.fori_loop(0, G, zb, 0)
    zbase = s * (NPAD // NS)
    for j in range(9):
        pltpu.async_copy(rows0_v, acc.at[pl.ds(zbase + j * G, G)], sem1)
    pltpu.async_copy(rows0_v.at[pl.ds(0, 50)], acc.at[pl.ds(zbase + 576, 50)],
                     sem1)
    for j in range(9):
        pltpu.make_async_copy(rows0_v, acc.at[pl.ds(zbase + j * G, G)],
                              sem1).wait()
    pltpu.make_async_copy(rows0_v.at[pl.ds(0, 50)],
                          acc.at[pl.ds(zbase + 576, 50)], sem1).wait()

    rho16 = rho_v[...]

    def judge(off):
        df = diff_v[pl.ds(off, L)]
        sv = src_v[pl.ds(off, L)]
        dv = dst_v[pl.ds(off, L)]
        keep = jnp.logical_and(
            plsc.bitcast(df, jnp.int32) >= 0, jnp.abs(df) < rho16)
        return sv, dv, keep, jnp.sum(keep.astype(jnp.int32))

    def emit(cnt, rec):
        sv, dv, keep, ksum = rec
        plsc.store_compressed(src_v.at[pl.ds(cnt, L)], sv, mask=keep)
        plsc.store_compressed(dst_v.at[pl.ds(cnt, L)], dv, mask=keep)
        return cnt + ksum

    UNROLL = 4

    def mbody(i, cnt):
        recs = [judge((i * UNROLL + u) * L) for u in range(UNROLL)]
        for rec in recs:
            cnt = emit(cnt, rec)
        return cnt

    NB = EPT // L
    cnt = lax.fori_loop(0, NB // UNROLL, mbody, 0)
    for u in range(NB - (NB // UNROLL) * UNROLL):
        cnt = emit(cnt, judge((NB - NB % UNROLL + u) * L))

    for j in range(G // L + 1):
        src_v[pl.ds(cnt + j * L, L)] = jnp.zeros((L,), jnp.int32)
        dst_v[pl.ds(cnt + j * L, L)] = jnp.full((L,), DUMMY, jnp.int32)
    plsc.subcore_barrier()

    nchunks = (cnt + G - 1) // G

    def gather(g, buf, sem):
        return pltpu.async_copy(
            feat_hbm.at[src_v.at[pl.ds(g * G, G)]], buf, sem)

    def drain(g, buf, idx, sem):
        pltpu.make_async_copy(
            feat_hbm.at[src_v.at[pl.ds(g * G, G)]], buf, sem).wait()

        def stage(j, carry2):
            idx[pl.ds(j * L, L)] = dst_v[pl.ds(g * G + j * L, L)]
            return carry2

        lax.fori_loop(0, G // L, stage, 0)
        pltpu.sync_copy(buf, acc.at[idx], add=True)

    @pl.when(nchunks > 0)
    def _():
        gather(0, rows0_v, sem0)

    def pair(p, carry):
        even = 2 * p
        odd = even + 1

        @pl.when(odd < nchunks)
        def _():
            gather(odd, rows1_v, sem1)

        drain(even, rows0_v, idx0_v, sem0)

        @pl.when(odd < nchunks)
        def _():
            @pl.when(odd + 1 < nchunks)
            def _():
                gather(odd + 1, rows0_v, sem0)

            drain(odd, rows1_v, idx1_v, sem1)

        return carry

    lax.fori_loop(0, (nchunks + 1) // 2, pair, 0)
    plsc.subcore_barrier()

    pltpu.async_copy(acc.at[pick_v], rows0_v, sem0).wait()
    pltpu.sync_copy(rows0_v, neigh_hbm.at[c, pl.ds(s * BPT, BPT)])


def _final_body(ctr_ref, neigh_ref, w_ref, out_ref):
    x = (ctr_ref[0] + ctr_ref[1]) * 0.5 + neigh_ref[0] + neigh_ref[1]
    y = lax.dot_general(x, w_ref[...], (((1,), (1,)), ((), ())),
                        preferred_element_type=jnp.float32)
    out_ref[...] = jnp.maximum(y, 0.0)


def _final(ctr, neigh, weight):
    return pl.pallas_call(
        _final_body,
        out_shape=jax.ShapeDtypeStruct((B, DF), jnp.float32),
    )(ctr, neigh, weight)


def kernel(features, weight, W1, b1, W2, b2, picked_nodes, edge_index):
    d = _dnet(features, W1, b1, W2, b2).reshape(R * N)
    edge_flat = edge_index.reshape(R * 2 * E)
    picked_flat = picked_nodes.reshape(R * B)
    diffs, ctr = _k2(d, edge_flat, picked_flat, features)
    rho = _rho(diffs.reshape(R, E)).reshape(R * 128)
    (neigh,) = _k4(rho, diffs, edge_flat, picked_flat, features)
    return _final(ctr, neigh, weight)
